# baseline (speedup 1.0000x reference)
import jax
import jax.numpy as jnp
from jax.experimental import pallas as pl
from jax.experimental.pallas import tpu as pltpu

DIM = 1024
NB = 8
MAB = 2
B = 8
H = DIM // 2


def _ln2d(x, s, b):
    mu = jnp.mean(x, axis=-1, keepdims=True)
    var = jnp.mean((x - mu) ** 2, axis=-1, keepdims=True)
    return (x - mu) / jnp.sqrt(var + 1e-5) * s + b


def _routing_body(xc_ref, xb_ref, wc_ref, bc_ref, wb_ref, bb_ref,
                  c_top_ref, b_top_ref):
    iota = jax.lax.broadcasted_iota(jnp.int32, (1, NB), 1)

    def top2(x_ref, w_ref, b_ref, out_ref):
        logits = jnp.dot(x_ref[:1], w_ref[:],
                         preferred_element_type=jnp.float32) + b_ref[:]
        sel = jax.nn.sigmoid(logits)
        adjusted = sel * 0.6 + 0.5 * 0.4
        m1 = jnp.max(adjusted)
        i1 = jnp.min(jnp.where(adjusted == m1, iota, NB))
        masked = jnp.where(iota == i1, -jnp.inf, adjusted)
        m2 = jnp.max(masked)
        i2 = jnp.min(jnp.where(masked == m2, iota, NB))
        out_ref[0] = i1
        out_ref[1] = i2

    top2(xc_ref, wc_ref, bc_ref, c_top_ref)
    top2(xb_ref, wb_ref, bb_ref, b_top_ref)


def _dot(a, b):
    return jnp.dot(a.astype(jnp.bfloat16), b.astype(jnp.bfloat16),
                   preferred_element_type=jnp.float32)


def _cortical_body(top_ref, x_ref, tors_ref, ps_ref, vecs_ref, ffb1_ref,
                   aW0_ref, aW1_ref, w10_ref, w11_ref, w20_ref, w21_ref,
                   out_ref):
    j = pl.program_id(0)

    @pl.when(j == 0)
    def _():
        out_ref[:] = x_ref[:]

    idx = top_ref[j]
    v = vecs_ref[idx]
    ln1s, ln1b, attnb = v[0:1], v[1:2], v[2:3]
    ln2s, ln2b, ffb2, gate = v[3:4], v[4:5], v[5:6], v[6:7]
    ffb1 = ffb1_ref[pl.ds(idx, 1), :]

    h_in = out_ref[:]
    fw = jax.nn.sigmoid(gate)
    h = _ln2d(h_in, ln1s, ln1b)
    h = jnp.concatenate([_dot(h, aW0_ref[0]), _dot(h, aW1_ref[0])], axis=-1)
    h = h + attnb
    h = h * (1.0 + fw * tors_ref[:])
    h = h_in + h * 0.5
    h2 = _ln2d(h, ln2s, ln2b)
    a0 = _dot(h2, w10_ref[0]) + ffb1[:, :DIM]
    a1 = _dot(h2, w11_ref[0]) + ffb1[:, DIM:]
    a0 = a0 * 0.5 * (1.0 + jax.lax.erf(a0 * (2.0 ** -0.5)))
    a1 = a1 * 0.5 * (1.0 + jax.lax.erf(a1 * (2.0 ** -0.5)))
    h2 = _dot(a0, w20_ref[0]) + _dot(a1, w21_ref[0]) + ffb2
    h2 = h2 + (ps_ref[:] * 0.3) * fw
    out_ref[:] = h + h2 * 0.5


def _brainstem_body(top_ref, x_ref, tors_ref, ps_ref, ch_ref, vecs_ref,
                    aW0_ref, aW1_ref, w10_ref, w11_ref, w20_ref, w21_ref,
                    cW0_ref, cW1_ref, crossb_ref,
                    out_ref, fused_ref):
    j = pl.program_id(0)

    @pl.when(j == 0)
    def _():
        out_ref[:] = x_ref[:]

    @pl.when(j < MAB)
    def _():
        idx = top_ref[jnp.minimum(j, MAB - 1)]
        v = vecs_ref[idx]
        ln1s, ln1b, attnb = v[0:1], v[1:2], v[2:3]
        ln2s, ln2b, ffb1, ffb2, gate = (v[3:4], v[4:5], v[5:6], v[6:7],
                                        v[7:8])

        h_in = out_ref[:]
        fw = jax.nn.sigmoid(gate)
        h = _ln2d(h_in, ln1s, ln1b)
        h = jnp.concatenate([_dot(h, aW0_ref[0]), _dot(h, aW1_ref[0])],
                            axis=-1)
        h = h + attnb
        h = h * (1.0 + fw * tors_ref[:])
        h = h_in + h * 0.5
        h2 = _ln2d(h, ln2s, ln2b)
        a0 = jnp.tanh(_dot(h2, w10_ref[0]) + ffb1[:, :H])
        a1 = jnp.tanh(_dot(h2, w11_ref[0]) + ffb1[:, H:])
        h2 = _dot(a0, w20_ref[0]) + _dot(a1, w21_ref[0]) + ffb2
        h2 = h2 + (ps_ref[:] * 0.3) * fw
        out_ref[:] = h + h2 * 0.5

    @pl.when(j == MAB)
    def _():
        fused = _dot(ch_ref[:], cW0_ref[0]) + _dot(out_ref[:], cW1_ref[0])
        fused_ref[:] = fused + crossb_ref[:]


def kernel(cortical_input, brainstem_input, torsion_field, params):
    xc = cortical_input.reshape(B, DIM)
    xb = brainstem_input.reshape(B, DIM)

    c_top, b_top = pl.pallas_call(
        _routing_body,
        out_shape=(jax.ShapeDtypeStruct((MAB,), jnp.int32),
                   jax.ShapeDtypeStruct((MAB,), jnp.int32)),
        in_specs=[
            pl.BlockSpec((B, DIM), lambda: (0, 0)),
            pl.BlockSpec((B, DIM), lambda: (0, 0)),
            pl.BlockSpec((DIM, NB), lambda: (0, 0)),
            pl.BlockSpec((1, NB), lambda: (0, 0)),
            pl.BlockSpec((DIM, NB), lambda: (0, 0)),
            pl.BlockSpec((1, NB), lambda: (0, 0)),
        ],
        out_specs=(pl.BlockSpec(memory_space=pltpu.SMEM),
                   pl.BlockSpec(memory_space=pltpu.SMEM)),
    )(xc, xb,
      params['sel_c_W'], params['sel_c_b'].reshape(1, NB),
      params['sel_b_W'], params['sel_b_b'].reshape(1, NB))

    fixed = lambda j, t: (0, 0)
    fixed3 = lambda j, t: (0, 0, 0)
    sel3 = lambda j, t: (t[j], 0, 0)
    sel3b = lambda j, t: (t[j], 0, 1)
    even3 = lambda j, t: (2 * t[j], 0, 0)
    odd3 = lambda j, t: (2 * t[j] + 1, 0, 0)

    pc = params['cortical']
    c_vecs = jnp.stack([pc['ln1_s'], pc['ln1_b'], pc['attn_b'],
                        pc['ln2_s'], pc['ln2_b'], pc['ff_b2'],
                        pc['gate']], axis=1)
    cortical_grid = pltpu.PrefetchScalarGridSpec(
        num_scalar_prefetch=1,
        grid=(MAB,),
        in_specs=[
            pl.BlockSpec((B, DIM), fixed),
            pl.BlockSpec((B, DIM), fixed),
            pl.BlockSpec((B, DIM), fixed),
            pl.BlockSpec((NB, 7, DIM), fixed3),
            pl.BlockSpec((NB, 2 * DIM), fixed),
            pl.BlockSpec((1, DIM, H), sel3),
            pl.BlockSpec((1, DIM, H), sel3b),
            pl.BlockSpec((1, DIM, DIM), sel3),
            pl.BlockSpec((1, DIM, DIM), sel3b),
            pl.BlockSpec((1, DIM, DIM), even3),
            pl.BlockSpec((1, DIM, DIM), odd3),
        ],
        out_specs=pl.BlockSpec((B, DIM), fixed),
    )
    cortical_h = pl.pallas_call(
        _cortical_body,
        grid_spec=cortical_grid,
        out_shape=jax.ShapeDtypeStruct((B, DIM), jnp.float32),
    )(c_top, xc, torsion_field, xb, c_vecs, pc['ff_b1'],
      pc['attn_W'], pc['attn_W'], pc['ff_W1'], pc['ff_W1'],
      pc['ff_W2'].reshape(2 * NB, DIM, DIM),
      pc['ff_W2'].reshape(2 * NB, DIM, DIM))

    pb = params['brainstem']
    b_vecs = jnp.stack([pb['ln1_s'], pb['ln1_b'], pb['attn_b'],
                        pb['ln2_s'], pb['ln2_b'], pb['ff_b1'],
                        pb['ff_b2'], pb['gate']], axis=1)
    clamp = lambda f: lambda j, t: f(jnp.minimum(j, MAB - 1), t)
    brainstem_grid = pltpu.PrefetchScalarGridSpec(
        num_scalar_prefetch=1,
        grid=(MAB + 1,),
        in_specs=[
            pl.BlockSpec((B, DIM), fixed),
            pl.BlockSpec((B, DIM), fixed),
            pl.BlockSpec((B, DIM), fixed),
            pl.BlockSpec((B, DIM), fixed),
            pl.BlockSpec((NB, 8, DIM), fixed3),
            pl.BlockSpec((1, DIM, H), clamp(sel3)),
            pl.BlockSpec((1, DIM, H), clamp(sel3b)),
            pl.BlockSpec((1, DIM, H), clamp(sel3)),
            pl.BlockSpec((1, DIM, H), clamp(sel3b)),
            pl.BlockSpec((1, H, DIM), clamp(even3)),
            pl.BlockSpec((1, H, DIM), clamp(odd3)),
            pl.BlockSpec((1, DIM, DIM), lambda j, t: (0, 0, 0)),
            pl.BlockSpec((1, DIM, DIM), lambda j, t: (1, 0, 0)),
            pl.BlockSpec((1, DIM), fixed),
        ],
        out_specs=(pl.BlockSpec((B, DIM), fixed),
                   pl.BlockSpec((B, DIM), fixed)),
    )
    brainstem_h, fused = pl.pallas_call(
        _brainstem_body,
        grid_spec=brainstem_grid,
        out_shape=(jax.ShapeDtypeStruct((B, DIM), jnp.float32),
                   jax.ShapeDtypeStruct((B, DIM), jnp.float32)),
    )(b_top, xb, torsion_field, xc, cortical_h, b_vecs,
      pb['attn_W'], pb['attn_W'], pb['ff_W1'], pb['ff_W1'],
      pb['ff_W2'].reshape(2 * NB, H, DIM),
      pb['ff_W2'].reshape(2 * NB, H, DIM),
      params['cross_W'].reshape(2, DIM, DIM),
      params['cross_W'].reshape(2, DIM, DIM),
      params['cross_b'].reshape(1, DIM))

    shape3 = (B, 1, DIM)
    return (cortical_h.reshape(shape3), brainstem_h.reshape(shape3),
            fused.reshape(shape3))


# speedup vs baseline: 1.0071x; 1.0071x over previous
import jax
import jax.numpy as jnp
from jax.experimental import pallas as pl
from jax.experimental.pallas import tpu as pltpu

DIM = 1024
NB = 8
MAB = 2
B = 8
H = DIM // 2


def _ln2d(x, s, b):
    mu = jnp.mean(x, axis=-1, keepdims=True)
    var = jnp.mean((x - mu) ** 2, axis=-1, keepdims=True)
    return (x - mu) / jnp.sqrt(var + 1e-5) * s + b


def _routing_body(xc_ref, xb_ref, wc_ref, bc_ref, wb_ref, bb_ref,
                  c_top_ref, b_top_ref):
    iota = jax.lax.broadcasted_iota(jnp.int32, (1, NB), 1)

    def top2(x_ref, w_ref, b_ref, out_ref):
        logits = jnp.dot(x_ref[:1], w_ref[:],
                         preferred_element_type=jnp.float32) + b_ref[:]
        sel = jax.nn.sigmoid(logits)
        adjusted = sel * 0.6 + 0.5 * 0.4
        m1 = jnp.max(adjusted)
        i1 = jnp.min(jnp.where(adjusted == m1, iota, NB))
        masked = jnp.where(iota == i1, -jnp.inf, adjusted)
        m2 = jnp.max(masked)
        i2 = jnp.min(jnp.where(masked == m2, iota, NB))
        out_ref[0] = i1
        out_ref[1] = i2

    top2(xc_ref, wc_ref, bc_ref, c_top_ref)
    top2(xb_ref, wb_ref, bb_ref, b_top_ref)


def _dot(a, b):
    return jnp.dot(a, b, preferred_element_type=jnp.float32)


def _cortical_body(top_ref, x_ref, tors_ref, ps_ref, vecs_ref, ffb1_ref,
                   aW0_ref, aW1_ref, w10_ref, w11_ref, w20_ref, w21_ref,
                   out_ref):
    j = pl.program_id(0)

    @pl.when(j == 0)
    def _():
        out_ref[:] = x_ref[:]

    idx = top_ref[j]
    v = vecs_ref[idx]
    ln1s, ln1b, attnb = v[0:1], v[1:2], v[2:3]
    ln2s, ln2b, ffb2, gate = v[3:4], v[4:5], v[5:6], v[6:7]
    ffb1 = ffb1_ref[pl.ds(idx, 1), :]

    h_in = out_ref[:]
    fw = jax.nn.sigmoid(gate)
    h = _ln2d(h_in, ln1s, ln1b)
    h = jnp.concatenate([_dot(h, aW0_ref[0]), _dot(h, aW1_ref[0])], axis=-1)
    h = h + attnb
    h = h * (1.0 + fw * tors_ref[:])
    h = h_in + h * 0.5
    h2 = _ln2d(h, ln2s, ln2b)
    a0 = _dot(h2, w10_ref[0]) + ffb1[:, :DIM]
    a1 = _dot(h2, w11_ref[0]) + ffb1[:, DIM:]
    a0 = a0 * 0.5 * (1.0 + jax.lax.erf(a0 * (2.0 ** -0.5)))
    a1 = a1 * 0.5 * (1.0 + jax.lax.erf(a1 * (2.0 ** -0.5)))
    h2 = _dot(a0, w20_ref[0]) + _dot(a1, w21_ref[0]) + ffb2
    h2 = h2 + (ps_ref[:] * 0.3) * fw
    out_ref[:] = h + h2 * 0.5


def _brainstem_body(top_ref, x_ref, tors_ref, ps_ref, ch_ref, vecs_ref,
                    aW0_ref, aW1_ref, w10_ref, w11_ref, w20_ref, w21_ref,
                    cW_ref, crossb_ref,
                    out_ref, fused_ref):
    j = pl.program_id(0)

    @pl.when(j == 0)
    def _():
        out_ref[:] = x_ref[:]

    @pl.when(j < MAB)
    def _():
        idx = top_ref[jnp.minimum(j, MAB - 1)]
        v = vecs_ref[idx]
        ln1s, ln1b, attnb = v[0:1], v[1:2], v[2:3]
        ln2s, ln2b, ffb1, ffb2, gate = (v[3:4], v[4:5], v[5:6], v[6:7],
                                        v[7:8])

        h_in = out_ref[:]
        fw = jax.nn.sigmoid(gate)
        h = _ln2d(h_in, ln1s, ln1b)
        h = jnp.concatenate([_dot(h, aW0_ref[0]), _dot(h, aW1_ref[0])],
                            axis=-1)
        h = h + attnb
        h = h * (1.0 + fw * tors_ref[:])
        h = h_in + h * 0.5
        h2 = _ln2d(h, ln2s, ln2b)
        a0 = jnp.tanh(_dot(h2, w10_ref[0]) + ffb1[:, :H])
        a1 = jnp.tanh(_dot(h2, w11_ref[0]) + ffb1[:, H:])
        h2 = _dot(a0, w20_ref[0]) + _dot(a1, w21_ref[0]) + ffb2
        h2 = h2 + (ps_ref[:] * 0.3) * fw
        out_ref[:] = h + h2 * 0.5

    @pl.when(j == 0)
    def _():
        fused_ref[:] = _dot(ch_ref[:], cW_ref[0]) + crossb_ref[:]

    @pl.when(j == MAB)
    def _():
        fused_ref[:] = fused_ref[:] + _dot(out_ref[:], cW_ref[0])


def kernel(cortical_input, brainstem_input, torsion_field, params):
    xc = cortical_input.reshape(B, DIM)
    xb = brainstem_input.reshape(B, DIM)

    c_top, b_top = pl.pallas_call(
        _routing_body,
        out_shape=(jax.ShapeDtypeStruct((MAB,), jnp.int32),
                   jax.ShapeDtypeStruct((MAB,), jnp.int32)),
        in_specs=[
            pl.BlockSpec((B, DIM), lambda: (0, 0)),
            pl.BlockSpec((B, DIM), lambda: (0, 0)),
            pl.BlockSpec((DIM, NB), lambda: (0, 0)),
            pl.BlockSpec((1, NB), lambda: (0, 0)),
            pl.BlockSpec((DIM, NB), lambda: (0, 0)),
            pl.BlockSpec((1, NB), lambda: (0, 0)),
        ],
        out_specs=(pl.BlockSpec(memory_space=pltpu.SMEM),
                   pl.BlockSpec(memory_space=pltpu.SMEM)),
    )(xc, xb,
      params['sel_c_W'], params['sel_c_b'].reshape(1, NB),
      params['sel_b_W'], params['sel_b_b'].reshape(1, NB))

    fixed = lambda j, t: (0, 0)
    fixed3 = lambda j, t: (0, 0, 0)
    sel3 = lambda j, t: (t[j], 0, 0)
    sel3b = lambda j, t: (t[j], 0, 1)
    even3 = lambda j, t: (2 * t[j], 0, 0)
    odd3 = lambda j, t: (2 * t[j] + 1, 0, 0)

    pc = params['cortical']
    c_vecs = jnp.stack([pc['ln1_s'], pc['ln1_b'], pc['attn_b'],
                        pc['ln2_s'], pc['ln2_b'], pc['ff_b2'],
                        pc['gate']], axis=1)
    cortical_grid = pltpu.PrefetchScalarGridSpec(
        num_scalar_prefetch=1,
        grid=(MAB,),
        in_specs=[
            pl.BlockSpec((B, DIM), fixed),
            pl.BlockSpec((B, DIM), fixed),
            pl.BlockSpec((B, DIM), fixed),
            pl.BlockSpec((NB, 7, DIM), fixed3),
            pl.BlockSpec((NB, 2 * DIM), fixed),
            pl.BlockSpec((1, DIM, H), sel3),
            pl.BlockSpec((1, DIM, H), sel3b),
            pl.BlockSpec((1, DIM, DIM), sel3),
            pl.BlockSpec((1, DIM, DIM), sel3b),
            pl.BlockSpec((1, DIM, DIM), even3),
            pl.BlockSpec((1, DIM, DIM), odd3),
        ],
        out_specs=pl.BlockSpec((B, DIM), fixed),
    )
    cortical_h = pl.pallas_call(
        _cortical_body,
        grid_spec=cortical_grid,
        out_shape=jax.ShapeDtypeStruct((B, DIM), jnp.float32),
    )(c_top, xc, torsion_field, xb, c_vecs, pc['ff_b1'],
      pc['attn_W'], pc['attn_W'], pc['ff_W1'], pc['ff_W1'],
      pc['ff_W2'].reshape(2 * NB, DIM, DIM),
      pc['ff_W2'].reshape(2 * NB, DIM, DIM))

    pb = params['brainstem']
    b_vecs = jnp.stack([pb['ln1_s'], pb['ln1_b'], pb['attn_b'],
                        pb['ln2_s'], pb['ln2_b'], pb['ff_b1'],
                        pb['ff_b2'], pb['gate']], axis=1)
    clamp = lambda f: lambda j, t: f(jnp.minimum(j, MAB - 1), t)
    brainstem_grid = pltpu.PrefetchScalarGridSpec(
        num_scalar_prefetch=1,
        grid=(MAB + 1,),
        in_specs=[
            pl.BlockSpec((B, DIM), fixed),
            pl.BlockSpec((B, DIM), fixed),
            pl.BlockSpec((B, DIM), fixed),
            pl.BlockSpec((B, DIM), fixed),
            pl.BlockSpec((NB, 8, DIM), fixed3),
            pl.BlockSpec((1, DIM, H), clamp(sel3)),
            pl.BlockSpec((1, DIM, H), clamp(sel3b)),
            pl.BlockSpec((1, DIM, H), clamp(sel3)),
            pl.BlockSpec((1, DIM, H), clamp(sel3b)),
            pl.BlockSpec((1, H, DIM), clamp(even3)),
            pl.BlockSpec((1, H, DIM), clamp(odd3)),
            pl.BlockSpec((1, DIM, DIM),
                         lambda j, t: (jnp.minimum(j, 1), 0, 0)),
            pl.BlockSpec((1, DIM), fixed),
        ],
        out_specs=(pl.BlockSpec((B, DIM), fixed),
                   pl.BlockSpec((B, DIM), fixed)),
    )
    brainstem_h, fused = pl.pallas_call(
        _brainstem_body,
        grid_spec=brainstem_grid,
        out_shape=(jax.ShapeDtypeStruct((B, DIM), jnp.float32),
                   jax.ShapeDtypeStruct((B, DIM), jnp.float32)),
    )(b_top, xb, torsion_field, xc, cortical_h, b_vecs,
      pb['attn_W'], pb['attn_W'], pb['ff_W1'], pb['ff_W1'],
      pb['ff_W2'].reshape(2 * NB, H, DIM),
      pb['ff_W2'].reshape(2 * NB, H, DIM),
      params['cross_W'].reshape(2, DIM, DIM),
      params['cross_b'].reshape(1, DIM))

    shape3 = (B, 1, DIM)
    return (cortical_h.reshape(shape3), brainstem_h.reshape(shape3),
            fused.reshape(shape3))
